# Initial kernel scaffold; baseline (speedup 1.0000x reference)
import jax
import jax.numpy as jnp
from jax.experimental import pallas as pl


def kernel(x, embed, first_cls):
    raise NotImplementedError("write your pallas kernel here")


# speedup vs baseline: 5.8267x; 5.8267x over previous
import functools

import jax
import jax.numpy as jnp
from jax import lax
from jax.experimental import pallas as pl
from jax.experimental.pallas import tpu as pltpu
from jax.experimental.pallas import tpu_sc as plsc

D_EMBED = 128
N_VOCAB = 101
BATCH = 4096
SEQ = 200
CHUNK = 100
N_CHUNK = SEQ // CHUNK


def _sc_embed(x2, embed, cls_row):
    info = plsc.get_sparse_core_info()
    nw = info.num_cores * info.num_subcores
    nb = BATCH // nw

    mesh = plsc.VectorSubcoreMesh(core_axis_name="c", subcore_axis_name="s")

    @functools.partial(
        pl.kernel,
        out_type=jax.ShapeDtypeStruct((BATCH, SEQ + 1, D_EMBED), jnp.float32),
        mesh=mesh,
        scratch_types=[
            pltpu.VMEM_SHARED((N_VOCAB, D_EMBED), jnp.float32),
            pltpu.VMEM((nb * N_CHUNK, CHUNK), jnp.int32),
            pltpu.VMEM((SEQ + 1, D_EMBED), jnp.float32),
            pltpu.VMEM((SEQ + 1, D_EMBED), jnp.float32),
            pltpu.SemaphoreType.DMA,
        ],
    )
    def k(x_hbm, tab_hbm, cls_hbm, out_hbm, tab_v, idx_v, buf_a, buf_b, gsem):
        sid = lax.axis_index("s")
        wid = sid * info.num_cores + lax.axis_index("c")

        @pl.when(sid == 0)
        def _():
            pltpu.sync_copy(tab_hbm, tab_v)
        pltpu.sync_copy(cls_hbm, buf_a.at[pl.ds(0, 1)])
        pltpu.sync_copy(cls_hbm, buf_b.at[pl.ds(0, 1)])
        plsc.subcore_barrier()
        pltpu.sync_copy(x_hbm.at[pl.ds(wid * (nb * N_CHUNK), nb * N_CHUNK)], idx_v)

        def do_batch(j, buf):
            b = wid * nb + j
            cps = []
            for c in range(N_CHUNK):
                cps.append(pltpu.async_copy(
                    tab_v.at[idx_v.at[j * N_CHUNK + c]],
                    buf.at[pl.ds(1 + c * CHUNK, CHUNK)],
                    gsem))
            for cp in cps:
                cp.wait()
            pltpu.sync_copy(buf, out_hbm.at[b])

        def body(i, carry):
            do_batch(2 * i, buf_a)
            do_batch(2 * i + 1, buf_b)
            return carry
        lax.fori_loop(0, nb // 2, body, 0)

    return k(x2, embed, cls_row)


def kernel(x, embed, first_cls):
    x2 = x.astype(jnp.int32).reshape(BATCH * N_CHUNK, CHUNK)
    cls_row = first_cls.reshape(1, D_EMBED)
    return _sc_embed(x2, embed, cls_row)
